# baseline (speedup 1.0000x reference)
import functools

import jax
import jax.numpy as jnp
from jax import lax
from jax.experimental import pallas as pl
from jax.experimental.pallas import tpu as pltpu
from jax.experimental.pallas import tpu_sc as plsc

N_PAD = 10240
NC, NS = 2, 16
NW = NC * NS
CHUNK = 128
BLK = 128
HIDDEN = 128


NBUF = 2


def _make_agg_wide(cpw):
    mesh = plsc.VectorSubcoreMesh(core_axis_name="c", subcore_axis_name="s")
    rows_per_tile = N_PAD // NS
    nz = rows_per_tile // CHUNK
    half = cpw // 2
    win = -(-(half + NBUF) // 8) * 8

    @functools.partial(
        pl.kernel,
        mesh=mesh,
        out_type=jax.ShapeDtypeStruct((NC, N_PAD, HIDDEN), jnp.float32),
        scratch_types=[
            pltpu.VMEM((win, CHUNK), jnp.int32),
            pltpu.VMEM((win, CHUNK), jnp.int32),
            pltpu.VMEM((NBUF, CHUNK, HIDDEN), jnp.float32),
            pltpu.VMEM_SHARED((N_PAD, HIDDEN), jnp.float32),
            pltpu.SemaphoreType.DMA((NBUF,)),
            pltpu.SemaphoreType.DMA((NBUF,)),
        ],
    )
    def agg(table_hbm, src_hbm, dst_hbm, zeros_hbm, out_hbm,
            src_v, dst_v, rows_v, acc_sh, gsem, ssem):
        c = lax.axis_index("c")
        s = lax.axis_index("s")
        wid = s * NC + c
        base = s * rows_per_tile
        for k in range(nz):
            pltpu.sync_copy(zeros_hbm, acc_sh.at[pl.ds(base + k * CHUNK, CHUNK)])
        plsc.subcore_barrier()

        def g_copy(j, b):
            return pltpu.make_async_copy(
                table_hbm.at[src_v.at[j]], rows_v.at[b], gsem.at[b])

        def s_copy(j, b):
            return pltpu.make_async_copy(
                rows_v.at[b], acc_sh.at[dst_v.at[j]], ssem.at[b])

        for h in range(2):
            pltpu.sync_copy(src_hbm.at[wid, pl.ds(h * half, win)], src_v)
            pltpu.sync_copy(dst_hbm.at[wid, pl.ds(h * half, win)], dst_v)
            for b in range(NBUF):
                g_copy(b, b).start()

            def body(jj, carry):
                j0 = jj * NBUF
                for b in range(NBUF):
                    g_copy(j0 + b, b).wait()
                    s_copy(j0 + b, b).start(add=True)
                for b in range(NBUF):
                    s_copy(j0 + b, b).wait()
                    g_copy(j0 + NBUF + b, b).start()
                return carry

            lax.fori_loop(0, half // NBUF, body, 0)
            for b in range(NBUF):
                g_copy(half + b, b).wait()

        plsc.subcore_barrier()
        pltpu.sync_copy(acc_sh.at[pl.ds(base, rows_per_tile)],
                        out_hbm.at[c, pl.ds(base, rows_per_tile)])

    return agg


def _row_spec(d):
    return pl.BlockSpec((BLK, d), lambda i: (i, 0))


def _full_spec(shape):
    return pl.BlockSpec(shape, lambda i: tuple(0 for _ in shape))


def _prep(h0, h1):
    def body(h0_ref, h1_ref, dinv_ref, a_ref):
        pid = pl.program_id(0)
        deg = h0_ref[...] + h1_ref[...] + 1.0
        dinv = lax.rsqrt(deg)
        rowid = (lax.broadcasted_iota(jnp.int32, (BLK, 1), 0)
                 + pid * BLK).astype(jnp.float32)
        dinv_ref[...] = dinv
        a_ref[...] = jnp.broadcast_to(dinv * rowid, (BLK, HIDDEN))

    return pl.pallas_call(
        body,
        grid=(N_PAD // BLK,),
        in_specs=[_row_spec(1), _row_spec(1)],
        out_specs=[_row_spec(1), _row_spec(HIDDEN)],
        out_shape=[jax.ShapeDtypeStruct((N_PAD, 1), jnp.float32),
                   jax.ShapeDtypeStruct((N_PAD, HIDDEN), jnp.float32)],
    )(h0, h1)


def _leaky(x):
    return jnp.where(x >= 0, x, 0.01 * x)


def _layer12(S0, S1, a, dinv, W1, b1, W2):
    def body(S0_ref, S1_ref, a_ref, dinv_ref, W1_ref, b1_ref, W2_ref, g2_ref):
        Ssum = S0_ref[...] + S1_ref[...] + a_ref[...]
        t = dinv_ref[...] * Ssum
        h1 = _leaky(t * W1_ref[...] + b1_ref[...])
        g2_ref[...] = dinv_ref[...] * jnp.dot(
            h1, W2_ref[...], preferred_element_type=jnp.float32)

    return pl.pallas_call(
        body,
        grid=(N_PAD // BLK,),
        in_specs=[_row_spec(1), _row_spec(1), _row_spec(1), _row_spec(1),
                  _full_spec((1, HIDDEN)), _full_spec((1, HIDDEN)),
                  _full_spec((HIDDEN, HIDDEN))],
        out_specs=_row_spec(HIDDEN),
        out_shape=jax.ShapeDtypeStruct((N_PAD, HIDDEN), jnp.float32),
    )(S0, S1, a, dinv, W1, b1, W2)


def _mid(P0, P1, g, dinv, b, W):
    def body(P0_ref, P1_ref, g_ref, dinv_ref, b_ref, W_ref, out_ref):
        agg = P0_ref[...] + P1_ref[...] + g_ref[...]
        h = _leaky(dinv_ref[...] * agg + b_ref[...])
        out_ref[...] = dinv_ref[...] * jnp.dot(
            h, W_ref[...], preferred_element_type=jnp.float32)

    return pl.pallas_call(
        body,
        grid=(N_PAD // BLK,),
        in_specs=[_row_spec(HIDDEN), _row_spec(HIDDEN), _row_spec(HIDDEN),
                  _row_spec(1), _full_spec((1, HIDDEN)),
                  _full_spec((HIDDEN, HIDDEN))],
        out_specs=_row_spec(HIDDEN),
        out_shape=jax.ShapeDtypeStruct((N_PAD, HIDDEN), jnp.float32),
    )(P0, P1, g, dinv, b, W)


def _final(P0, P1, g, dinv, b, d_out):
    def body(P0_ref, P1_ref, g_ref, dinv_ref, b_ref, out_ref):
        agg = (P0_ref[...] + P1_ref[...] + g_ref[...])[:, :d_out]
        z = dinv_ref[...] * agg + b_ref[...]
        m = jnp.max(z, axis=1, keepdims=True)
        e = jnp.exp(z - m)
        out_ref[...] = e / jnp.sum(e, axis=1, keepdims=True)

    return pl.pallas_call(
        body,
        grid=(N_PAD // BLK,),
        in_specs=[_row_spec(HIDDEN), _row_spec(HIDDEN), _row_spec(HIDDEN),
                  _row_spec(1), _full_spec((1, d_out))],
        out_specs=pl.BlockSpec((BLK, d_out), lambda i: (i, 0)),
        out_shape=jax.ShapeDtypeStruct((N_PAD, d_out), jnp.float32),
    )(P0, P1, g, dinv, b)


def kernel(x, edge_index, W1, b1, W2, b2, W3, b3, W4, b4):
    n = x.shape[0]
    e = edge_index.shape[1]
    n_classes = W4.shape[1]

    src = edge_index[0].astype(jnp.int32)
    dst = edge_index[1].astype(jnp.int32)
    cpw = -(-e // (NW * CHUNK))
    cpw = -(-cpw // 16) * 16
    win = -(-(cpw // 2 + NBUF) // 8) * 8
    cpx = cpw // 2 + win
    tot = NW * cpw * CHUNK
    src_p = jnp.concatenate([src, jnp.full((tot - e,), n, jnp.int32)])
    dst_p = jnp.concatenate([dst, jnp.full((tot - e,), n + 16, jnp.int32)])
    pf_s = jnp.full((NW, cpx - cpw, CHUNK), n, jnp.int32)
    pf_d = jnp.full((NW, cpx - cpw, CHUNK), n + 16, jnp.int32)
    src3 = jnp.concatenate([src_p.reshape(NW, cpw, CHUNK), pf_s], axis=1)
    dst3 = jnp.concatenate([dst_p.reshape(NW, cpw, CHUNK), pf_d], axis=1)

    z128 = jnp.zeros((CHUNK, HIDDEN), jnp.float32)
    ones = jnp.ones((N_PAD, HIDDEN), jnp.float32)
    b1r, b2r, b3r = (b.reshape(1, -1) for b in (b1, b2, b3))
    b4r = b4.reshape(1, -1)
    W4p = jnp.pad(W4, ((0, 0), (0, HIDDEN - n_classes)))

    agg_w = _make_agg_wide(cpw)

    hist = agg_w(ones, src3, dst3, z128)
    dinv, a = _prep(hist[0, :, :1], hist[1, :, :1])
    S = agg_w(a, src3, dst3, z128)
    g2 = _layer12(S[0, :, :1], S[1, :, :1], a[:, :1], dinv, W1, b1r, W2)
    P = agg_w(g2, src3, dst3, z128)
    g3 = _mid(P[0], P[1], g2, dinv, b2r, W3)
    P = agg_w(g3, src3, dst3, z128)
    g4 = _mid(P[0], P[1], g3, dinv, b3r, W4p)
    P = agg_w(g4, src3, dst3, z128)
    out = _final(P[0], P[1], g4, dinv, b4r, n_classes)
    return out[:n]


# speedup vs baseline: 1.0128x; 1.0128x over previous
import functools

import jax
import jax.numpy as jnp
from jax import lax
from jax.experimental import pallas as pl
from jax.experimental.pallas import tpu as pltpu
from jax.experimental.pallas import tpu_sc as plsc

N_PAD = 10240
NC, NS = 2, 16
NW = NC * NS
CHUNK = 128
BLK = 128
HIDDEN = 128


NBUF = 2


def _make_agg_wide(cpw):
    mesh = plsc.VectorSubcoreMesh(core_axis_name="c", subcore_axis_name="s")
    rows_per_tile = N_PAD // NS
    nz = rows_per_tile // CHUNK
    half = cpw // 2
    win = -(-(half + NBUF) // 8) * 8

    @functools.partial(
        pl.kernel,
        mesh=mesh,
        out_type=jax.ShapeDtypeStruct((NC, N_PAD, HIDDEN), jnp.float32),
        scratch_types=[
            pltpu.VMEM((win, CHUNK), jnp.int32),
            pltpu.VMEM((win, CHUNK), jnp.int32),
            pltpu.VMEM((NBUF, CHUNK, HIDDEN), jnp.float32),
            pltpu.VMEM_SHARED((N_PAD, HIDDEN), jnp.float32),
            pltpu.SemaphoreType.DMA((NBUF,)),
            pltpu.SemaphoreType.DMA((NBUF,)),
        ],
    )
    def agg(table_hbm, src_hbm, dst_hbm, zeros_hbm, out_hbm,
            src_v, dst_v, rows_v, acc_sh, gsem, ssem):
        c = lax.axis_index("c")
        s = lax.axis_index("s")
        wid = s * NC + c
        base = s * rows_per_tile
        for k in range(nz):
            pltpu.sync_copy(zeros_hbm, acc_sh.at[pl.ds(base + k * CHUNK, CHUNK)])
        plsc.subcore_barrier()

        def g_copy(j, b):
            return pltpu.make_async_copy(
                table_hbm.at[src_v.at[j]], rows_v.at[b], gsem.at[b])

        def s_copy(j, b):
            return pltpu.make_async_copy(
                rows_v.at[b], acc_sh.at[dst_v.at[j]], ssem.at[b])

        for h in range(2):
            pltpu.sync_copy(src_hbm.at[wid, pl.ds(h * half, win)], src_v)
            pltpu.sync_copy(dst_hbm.at[wid, pl.ds(h * half, win)], dst_v)
            for b in range(NBUF):
                g_copy(b, b).start()

            def body(jj, carry):
                j0 = jj * NBUF
                for b in range(NBUF):
                    g_copy(j0 + b, b).wait()
                    pltpu.sync_copy(rows_v.at[b],
                                    acc_sh.at[dst_v.at[j0 + b]], add=True)
                    g_copy(j0 + NBUF + b, b).start()
                return carry

            lax.fori_loop(0, half // NBUF, body, 0)
            for b in range(NBUF):
                g_copy(half + b, b).wait()

        plsc.subcore_barrier()
        pltpu.sync_copy(acc_sh.at[pl.ds(base, rows_per_tile)],
                        out_hbm.at[c, pl.ds(base, rows_per_tile)])

    return agg


def _row_spec(d):
    return pl.BlockSpec((BLK, d), lambda i: (i, 0))


def _full_spec(shape):
    return pl.BlockSpec(shape, lambda i: tuple(0 for _ in shape))


def _prep(h0, h1):
    def body(h0_ref, h1_ref, dinv_ref, a_ref):
        pid = pl.program_id(0)
        deg = h0_ref[...] + h1_ref[...] + 1.0
        dinv = lax.rsqrt(deg)
        rowid = (lax.broadcasted_iota(jnp.int32, (BLK, 1), 0)
                 + pid * BLK).astype(jnp.float32)
        dinv_ref[...] = dinv
        a_ref[...] = jnp.broadcast_to(dinv * rowid, (BLK, HIDDEN))

    return pl.pallas_call(
        body,
        grid=(N_PAD // BLK,),
        in_specs=[_row_spec(1), _row_spec(1)],
        out_specs=[_row_spec(1), _row_spec(HIDDEN)],
        out_shape=[jax.ShapeDtypeStruct((N_PAD, 1), jnp.float32),
                   jax.ShapeDtypeStruct((N_PAD, HIDDEN), jnp.float32)],
    )(h0, h1)


def _leaky(x):
    return jnp.where(x >= 0, x, 0.01 * x)


def _layer12(S0, S1, a, dinv, W1, b1, W2):
    def body(S0_ref, S1_ref, a_ref, dinv_ref, W1_ref, b1_ref, W2_ref, g2_ref):
        Ssum = S0_ref[...] + S1_ref[...] + a_ref[...]
        t = dinv_ref[...] * Ssum
        h1 = _leaky(t * W1_ref[...] + b1_ref[...])
        g2_ref[...] = dinv_ref[...] * jnp.dot(
            h1, W2_ref[...], preferred_element_type=jnp.float32)

    return pl.pallas_call(
        body,
        grid=(N_PAD // BLK,),
        in_specs=[_row_spec(1), _row_spec(1), _row_spec(1), _row_spec(1),
                  _full_spec((1, HIDDEN)), _full_spec((1, HIDDEN)),
                  _full_spec((HIDDEN, HIDDEN))],
        out_specs=_row_spec(HIDDEN),
        out_shape=jax.ShapeDtypeStruct((N_PAD, HIDDEN), jnp.float32),
    )(S0, S1, a, dinv, W1, b1, W2)


def _mid(P0, P1, g, dinv, b, W):
    def body(P0_ref, P1_ref, g_ref, dinv_ref, b_ref, W_ref, out_ref):
        agg = P0_ref[...] + P1_ref[...] + g_ref[...]
        h = _leaky(dinv_ref[...] * agg + b_ref[...])
        out_ref[...] = dinv_ref[...] * jnp.dot(
            h, W_ref[...], preferred_element_type=jnp.float32)

    return pl.pallas_call(
        body,
        grid=(N_PAD // BLK,),
        in_specs=[_row_spec(HIDDEN), _row_spec(HIDDEN), _row_spec(HIDDEN),
                  _row_spec(1), _full_spec((1, HIDDEN)),
                  _full_spec((HIDDEN, HIDDEN))],
        out_specs=_row_spec(HIDDEN),
        out_shape=jax.ShapeDtypeStruct((N_PAD, HIDDEN), jnp.float32),
    )(P0, P1, g, dinv, b, W)


def _final(P0, P1, g, dinv, b, d_out):
    def body(P0_ref, P1_ref, g_ref, dinv_ref, b_ref, out_ref):
        agg = (P0_ref[...] + P1_ref[...] + g_ref[...])[:, :d_out]
        z = dinv_ref[...] * agg + b_ref[...]
        m = jnp.max(z, axis=1, keepdims=True)
        e = jnp.exp(z - m)
        out_ref[...] = e / jnp.sum(e, axis=1, keepdims=True)

    return pl.pallas_call(
        body,
        grid=(N_PAD // BLK,),
        in_specs=[_row_spec(HIDDEN), _row_spec(HIDDEN), _row_spec(HIDDEN),
                  _row_spec(1), _full_spec((1, d_out))],
        out_specs=pl.BlockSpec((BLK, d_out), lambda i: (i, 0)),
        out_shape=jax.ShapeDtypeStruct((N_PAD, d_out), jnp.float32),
    )(P0, P1, g, dinv, b)


def kernel(x, edge_index, W1, b1, W2, b2, W3, b3, W4, b4):
    n = x.shape[0]
    e = edge_index.shape[1]
    n_classes = W4.shape[1]

    src = edge_index[0].astype(jnp.int32)
    dst = edge_index[1].astype(jnp.int32)
    cpw = -(-e // (NW * CHUNK))
    cpw = -(-cpw // 16) * 16
    win = -(-(cpw // 2 + NBUF) // 8) * 8
    cpx = cpw // 2 + win
    tot = NW * cpw * CHUNK
    src_p = jnp.concatenate([src, jnp.full((tot - e,), n, jnp.int32)])
    dst_p = jnp.concatenate([dst, jnp.full((tot - e,), n + 16, jnp.int32)])
    pf_s = jnp.full((NW, cpx - cpw, CHUNK), n, jnp.int32)
    pf_d = jnp.full((NW, cpx - cpw, CHUNK), n + 16, jnp.int32)
    src3 = jnp.concatenate([src_p.reshape(NW, cpw, CHUNK), pf_s], axis=1)
    dst3 = jnp.concatenate([dst_p.reshape(NW, cpw, CHUNK), pf_d], axis=1)

    z128 = jnp.zeros((CHUNK, HIDDEN), jnp.float32)
    ones = jnp.ones((N_PAD, HIDDEN), jnp.float32)
    b1r, b2r, b3r = (b.reshape(1, -1) for b in (b1, b2, b3))
    b4r = b4.reshape(1, -1)
    W4p = jnp.pad(W4, ((0, 0), (0, HIDDEN - n_classes)))

    agg_w = _make_agg_wide(cpw)

    hist = agg_w(ones, src3, dst3, z128)
    dinv, a = _prep(hist[0, :, :1], hist[1, :, :1])
    S = agg_w(a, src3, dst3, z128)
    g2 = _layer12(S[0, :, :1], S[1, :, :1], a[:, :1], dinv, W1, b1r, W2)
    P = agg_w(g2, src3, dst3, z128)
    g3 = _mid(P[0], P[1], g2, dinv, b2r, W3)
    P = agg_w(g3, src3, dst3, z128)
    g4 = _mid(P[0], P[1], g3, dinv, b3r, W4p)
    P = agg_w(g4, src3, dst3, z128)
    out = _final(P[0], P[1], g4, dinv, b4r, n_classes)
    return out[:n]


# speedup vs baseline: 1.6059x; 1.5856x over previous
import functools

import jax
import jax.numpy as jnp
from jax import lax
from jax.experimental import pallas as pl
from jax.experimental.pallas import tpu as pltpu
from jax.experimental.pallas import tpu_sc as plsc

N_PAD = 10240
NC, NS = 2, 16
NW = NC * NS
CHUNK = 128
BLK = 128
HIDDEN = 128


BIG = 256
L = 16


def _make_agg_wide(cpw):
    mesh = plsc.VectorSubcoreMesh(core_axis_name="c", subcore_axis_name="s")
    rows_per_tile = N_PAD // NS
    nz = rows_per_tile // CHUNK
    ept = cpw * CHUNK
    nb = ept // BIG

    @functools.partial(
        pl.kernel,
        mesh=mesh,
        out_type=jax.ShapeDtypeStruct((NC, N_PAD, HIDDEN), jnp.float32),
        scratch_types=[
            pltpu.VMEM((ept,), jnp.int32),
            pltpu.VMEM((1, BIG), jnp.int32),
            pltpu.VMEM((1, BIG), jnp.int32),
            pltpu.VMEM((BIG, HIDDEN), jnp.float32),
            pltpu.VMEM_SHARED((N_PAD, HIDDEN), jnp.float32),
        ],
    )
    def agg(table_hbm, packed_hbm, zeros_hbm, out_hbm,
            packed_v, sbuf, dbuf, rows_v, acc_sh):
        c = lax.axis_index("c")
        s = lax.axis_index("s")
        wid = s * NC + c
        base = s * rows_per_tile
        pltpu.sync_copy(packed_hbm.at[wid], packed_v)
        for k in range(nz):
            pltpu.sync_copy(zeros_hbm, acc_sh.at[pl.ds(base + k * CHUNK, CHUNK)])
        plsc.subcore_barrier()

        def body(j, carry):
            for k in range(BIG // L):
                p = packed_v[pl.ds(j * BIG + k * L, L)]
                sbuf[0, pl.ds(k * L, L)] = jnp.bitwise_and(p, 16383)
                dbuf[0, pl.ds(k * L, L)] = lax.shift_right_logical(p, 14)
            pltpu.sync_copy(table_hbm.at[sbuf.at[0]], rows_v)
            pltpu.sync_copy(rows_v, acc_sh.at[dbuf.at[0]], add=True)
            return carry

        lax.fori_loop(0, nb, body, 0)
        plsc.subcore_barrier()
        pltpu.sync_copy(acc_sh.at[pl.ds(base, rows_per_tile)],
                        out_hbm.at[c, pl.ds(base, rows_per_tile)])

    return agg


def _row_spec(d):
    return pl.BlockSpec((BLK, d), lambda i: (i, 0))


def _full_spec(shape):
    return pl.BlockSpec(shape, lambda i: tuple(0 for _ in shape))


def _prep(h0, h1):
    def body(h0_ref, h1_ref, dinv_ref, a_ref):
        pid = pl.program_id(0)
        deg = h0_ref[...] + h1_ref[...] + 1.0
        dinv = lax.rsqrt(deg)
        rowid = (lax.broadcasted_iota(jnp.int32, (BLK, 1), 0)
                 + pid * BLK).astype(jnp.float32)
        dinv_ref[...] = dinv
        a_ref[...] = jnp.broadcast_to(dinv * rowid, (BLK, HIDDEN))

    return pl.pallas_call(
        body,
        grid=(N_PAD // BLK,),
        in_specs=[_row_spec(1), _row_spec(1)],
        out_specs=[_row_spec(1), _row_spec(HIDDEN)],
        out_shape=[jax.ShapeDtypeStruct((N_PAD, 1), jnp.float32),
                   jax.ShapeDtypeStruct((N_PAD, HIDDEN), jnp.float32)],
    )(h0, h1)


def _leaky(x):
    return jnp.where(x >= 0, x, 0.01 * x)


def _layer12(S0, S1, a, dinv, W1, b1, W2):
    def body(S0_ref, S1_ref, a_ref, dinv_ref, W1_ref, b1_ref, W2_ref, g2_ref):
        Ssum = S0_ref[...] + S1_ref[...] + a_ref[...]
        t = dinv_ref[...] * Ssum
        h1 = _leaky(t * W1_ref[...] + b1_ref[...])
        g2_ref[...] = dinv_ref[...] * jnp.dot(
            h1, W2_ref[...], preferred_element_type=jnp.float32)

    return pl.pallas_call(
        body,
        grid=(N_PAD // BLK,),
        in_specs=[_row_spec(1), _row_spec(1), _row_spec(1), _row_spec(1),
                  _full_spec((1, HIDDEN)), _full_spec((1, HIDDEN)),
                  _full_spec((HIDDEN, HIDDEN))],
        out_specs=_row_spec(HIDDEN),
        out_shape=jax.ShapeDtypeStruct((N_PAD, HIDDEN), jnp.float32),
    )(S0, S1, a, dinv, W1, b1, W2)


def _mid(P0, P1, g, dinv, b, W):
    def body(P0_ref, P1_ref, g_ref, dinv_ref, b_ref, W_ref, out_ref):
        agg = P0_ref[...] + P1_ref[...] + g_ref[...]
        h = _leaky(dinv_ref[...] * agg + b_ref[...])
        out_ref[...] = dinv_ref[...] * jnp.dot(
            h, W_ref[...], preferred_element_type=jnp.float32)

    return pl.pallas_call(
        body,
        grid=(N_PAD // BLK,),
        in_specs=[_row_spec(HIDDEN), _row_spec(HIDDEN), _row_spec(HIDDEN),
                  _row_spec(1), _full_spec((1, HIDDEN)),
                  _full_spec((HIDDEN, HIDDEN))],
        out_specs=_row_spec(HIDDEN),
        out_shape=jax.ShapeDtypeStruct((N_PAD, HIDDEN), jnp.float32),
    )(P0, P1, g, dinv, b, W)


def _final(P0, P1, g, dinv, b, d_out):
    def body(P0_ref, P1_ref, g_ref, dinv_ref, b_ref, out_ref):
        agg = (P0_ref[...] + P1_ref[...] + g_ref[...])[:, :d_out]
        z = dinv_ref[...] * agg + b_ref[...]
        m = jnp.max(z, axis=1, keepdims=True)
        e = jnp.exp(z - m)
        out_ref[...] = e / jnp.sum(e, axis=1, keepdims=True)

    return pl.pallas_call(
        body,
        grid=(N_PAD // BLK,),
        in_specs=[_row_spec(HIDDEN), _row_spec(HIDDEN), _row_spec(HIDDEN),
                  _row_spec(1), _full_spec((1, d_out))],
        out_specs=pl.BlockSpec((BLK, d_out), lambda i: (i, 0)),
        out_shape=jax.ShapeDtypeStruct((N_PAD, d_out), jnp.float32),
    )(P0, P1, g, dinv, b)


def kernel(x, edge_index, W1, b1, W2, b2, W3, b3, W4, b4):
    n = x.shape[0]
    e = edge_index.shape[1]
    n_classes = W4.shape[1]

    src = edge_index[0].astype(jnp.int32)
    dst = edge_index[1].astype(jnp.int32)
    cpw = -(-e // (NW * CHUNK))
    cpw = -(-cpw // (2 * BIG // CHUNK)) * (2 * BIG // CHUNK)
    tot = NW * cpw * CHUNK
    src_p = jnp.concatenate([src, jnp.full((tot - e,), n, jnp.int32)])
    dst_p = jnp.concatenate([dst, jnp.full((tot - e,), n + 16, jnp.int32)])
    packed = (dst_p * 16384 + src_p).reshape(NW, cpw * CHUNK)

    z128 = jnp.zeros((CHUNK, HIDDEN), jnp.float32)
    ones = jnp.ones((N_PAD, HIDDEN), jnp.float32)
    b1r, b2r, b3r = (b.reshape(1, -1) for b in (b1, b2, b3))
    b4r = b4.reshape(1, -1)
    W4p = jnp.pad(W4, ((0, 0), (0, HIDDEN - n_classes)))

    agg_w = _make_agg_wide(cpw)

    hist = agg_w(ones, packed, z128)
    dinv, a = _prep(hist[0, :, :1], hist[1, :, :1])
    S = agg_w(a, packed, z128)
    g2 = _layer12(S[0, :, :1], S[1, :, :1], a[:, :1], dinv, W1, b1r, W2)
    P = agg_w(g2, packed, z128)
    g3 = _mid(P[0], P[1], g2, dinv, b2r, W3)
    P = agg_w(g3, packed, z128)
    g4 = _mid(P[0], P[1], g3, dinv, b3r, W4p)
    P = agg_w(g4, packed, z128)
    out = _final(P[0], P[1], g4, dinv, b4r, n_classes)
    return out[:n]


# speedup vs baseline: 3.5915x; 2.2364x over previous
import functools

import jax
import jax.numpy as jnp
from jax import lax
from jax.experimental import pallas as pl
from jax.experimental.pallas import tpu as pltpu
from jax.experimental.pallas import tpu_sc as plsc

N_PAD = 10240
NC, NS = 2, 16
NW = NC * NS
CHUNK = 128
BLK = 128
HIDDEN = 128


def _make_agg(cpw, d, tc_tiling=True):
    mesh = plsc.VectorSubcoreMesh(core_axis_name="c", subcore_axis_name="s")
    rows_per_tile = N_PAD // NS
    nz = rows_per_tile // CHUNK

    @functools.partial(
        pl.kernel,
        mesh=mesh,
        out_type=jax.ShapeDtypeStruct((NC, N_PAD, d), jnp.float32),
        scratch_types=[
            pltpu.VMEM((cpw, CHUNK), jnp.int32),
            pltpu.VMEM((cpw, CHUNK), jnp.int32),
            pltpu.VMEM((CHUNK, d), jnp.float32),
            pltpu.VMEM_SHARED((N_PAD, d), jnp.float32),
        ],
        compiler_params=pltpu.CompilerParams(use_tc_tiling_on_sc=tc_tiling),
    )
    def agg(table_hbm, src_hbm, dst_hbm, zeros_hbm, out_hbm,
            src_v, dst_v, rows_v, acc_sh):
        c = lax.axis_index("c")
        s = lax.axis_index("s")
        wid = s * NC + c
        base = s * rows_per_tile
        pltpu.sync_copy(src_hbm.at[wid], src_v)
        pltpu.sync_copy(dst_hbm.at[wid], dst_v)
        for k in range(nz):
            pltpu.sync_copy(zeros_hbm, acc_sh.at[pl.ds(base + k * CHUNK, CHUNK)])
        plsc.subcore_barrier()

        def body(j, carry):
            pltpu.sync_copy(table_hbm.at[src_v.at[j]], rows_v)
            pltpu.sync_copy(rows_v, acc_sh.at[dst_v.at[j]], add=True)
            return carry

        lax.fori_loop(0, cpw, body, 0)
        plsc.subcore_barrier()
        pltpu.sync_copy(acc_sh.at[pl.ds(base, rows_per_tile)],
                        out_hbm.at[c, pl.ds(base, rows_per_tile)])

    return agg


def _row_spec(d):
    return pl.BlockSpec((BLK, d), lambda i: (i, 0))


def _full_spec(shape):
    return pl.BlockSpec(shape, lambda i: tuple(0 for _ in shape))


def _prep(h0, h1):
    def body(h0_ref, h1_ref, dinv_ref, a_ref):
        pid = pl.program_id(0)
        deg = h0_ref[...] + h1_ref[...] + 1.0
        dinv = lax.rsqrt(deg)
        rowid = (lax.broadcasted_iota(jnp.int32, (BLK, 1), 0)
                 + pid * BLK).astype(jnp.float32)
        dinv_ref[...] = dinv
        a_ref[...] = jnp.broadcast_to(dinv * rowid, (BLK, 16))

    return pl.pallas_call(
        body,
        grid=(N_PAD // BLK,),
        in_specs=[_row_spec(1), _row_spec(1)],
        out_specs=[_row_spec(1), _row_spec(16)],
        out_shape=[jax.ShapeDtypeStruct((N_PAD, 1), jnp.float32),
                   jax.ShapeDtypeStruct((N_PAD, 16), jnp.float32)],
    )(h0, h1)


def _leaky(x):
    return jnp.where(x >= 0, x, 0.01 * x)


def _layer12(S0, S1, a, dinv, W1, b1, W2):
    def body(S0_ref, S1_ref, a_ref, dinv_ref, W1_ref, b1_ref, W2_ref, g2_ref):
        Ssum = S0_ref[...] + S1_ref[...] + a_ref[...]
        t = dinv_ref[...] * Ssum
        h1 = _leaky(t * W1_ref[...] + b1_ref[...])
        g2_ref[...] = dinv_ref[...] * jnp.dot(
            h1, W2_ref[...], preferred_element_type=jnp.float32)

    return pl.pallas_call(
        body,
        grid=(N_PAD // BLK,),
        in_specs=[_row_spec(1), _row_spec(1), _row_spec(1), _row_spec(1),
                  _full_spec((1, HIDDEN)), _full_spec((1, HIDDEN)),
                  _full_spec((HIDDEN, HIDDEN))],
        out_specs=_row_spec(HIDDEN),
        out_shape=jax.ShapeDtypeStruct((N_PAD, HIDDEN), jnp.float32),
    )(S0, S1, a, dinv, W1, b1, W2)


def _mid(P0, P1, g, dinv, b, W, d_out):
    def body(P0_ref, P1_ref, g_ref, dinv_ref, b_ref, W_ref, out_ref):
        agg = P0_ref[...] + P1_ref[...] + g_ref[...]
        h = _leaky(dinv_ref[...] * agg + b_ref[...])
        out_ref[...] = dinv_ref[...] * jnp.dot(
            h, W_ref[...], preferred_element_type=jnp.float32)

    return pl.pallas_call(
        body,
        grid=(N_PAD // BLK,),
        in_specs=[_row_spec(HIDDEN), _row_spec(HIDDEN), _row_spec(HIDDEN),
                  _row_spec(1), _full_spec((1, HIDDEN)),
                  _full_spec((HIDDEN, d_out))],
        out_specs=_row_spec(d_out),
        out_shape=jax.ShapeDtypeStruct((N_PAD, d_out), jnp.float32),
    )(P0, P1, g, dinv, b, W)


def _final(P0, P1, g, dinv, b, d_out):
    def body(P0_ref, P1_ref, g_ref, dinv_ref, b_ref, out_ref):
        z = dinv_ref[...] * (P0_ref[...] + P1_ref[...] + g_ref[...]) + b_ref[...]
        m = jnp.max(z, axis=1, keepdims=True)
        e = jnp.exp(z - m)
        out_ref[...] = e / jnp.sum(e, axis=1, keepdims=True)

    return pl.pallas_call(
        body,
        grid=(N_PAD // BLK,),
        in_specs=[_row_spec(d_out), _row_spec(d_out), _row_spec(d_out),
                  _row_spec(1), _full_spec((1, d_out))],
        out_specs=_row_spec(d_out),
        out_shape=jax.ShapeDtypeStruct((N_PAD, d_out), jnp.float32),
    )(P0, P1, g, dinv, b)


def kernel(x, edge_index, W1, b1, W2, b2, W3, b3, W4, b4):
    n = x.shape[0]
    e = edge_index.shape[1]
    n_classes = W4.shape[1]

    src = edge_index[0].astype(jnp.int32)
    dst = edge_index[1].astype(jnp.int32)
    cpw = -(-e // (NW * CHUNK))
    tot = NW * cpw * CHUNK
    src_p = jnp.concatenate([src, jnp.full((tot - e,), n, jnp.int32)])
    dst_p = jnp.concatenate([dst, jnp.full((tot - e,), n + 16, jnp.int32)])
    src3 = src_p.reshape(NW, cpw, CHUNK)
    dst3 = dst_p.reshape(NW, cpw, CHUNK)

    z16 = jnp.zeros((CHUNK, n_classes), jnp.float32)
    z128 = jnp.zeros((CHUNK, HIDDEN), jnp.float32)
    ones16 = jnp.ones((N_PAD, n_classes), jnp.float32)
    b1r, b2r, b3r = (b.reshape(1, -1) for b in (b1, b2, b3))
    b4r = b4.reshape(1, -1)

    agg16 = _make_agg(cpw, n_classes, tc_tiling=False)
    agg128 = _make_agg(cpw, HIDDEN)

    hist = agg16(ones16, src3, dst3, z16)
    dinv, a = _prep(hist[0, :, :1], hist[1, :, :1])
    S = agg16(a, src3, dst3, z16)
    g2 = _layer12(S[0, :, :1], S[1, :, :1], a[:, :1], dinv, W1, b1r, W2)
    P = agg128(g2, src3, dst3, z128)
    g3 = _mid(P[0], P[1], g2, dinv, b2r, W3, HIDDEN)
    P = agg128(g3, src3, dst3, z128)
    g4 = _mid(P[0], P[1], g3, dinv, b3r, W4, n_classes)
    P = agg16(g4, src3, dst3, z16)
    out = _final(P[0], P[1], g4, dinv, b4r, n_classes)
    return out[:n]
